# speedup vs baseline: 2.2790x; 2.2790x over previous
import functools

import jax
import jax.numpy as jnp
from jax import lax
from jax.experimental import pallas as pl
from jax.experimental.pallas import tpu as pltpu
from jax.experimental.pallas import tpu_sc as plsc

SEQ_LEN = 8192
D_MODEL = 768
BATCH = 4

NB = BATCH * SEQ_LEN
NC = 2
NS = 16
NW = NC * NS
BPW = NB // NW
CH = 64
NCHUNK = BPW // CH

_mesh = plsc.VectorSubcoreMesh(core_axis_name="c", subcore_axis_name="s")


@functools.partial(
    pl.kernel,
    out_type=jax.ShapeDtypeStruct((NB, D_MODEL), jnp.float32),
    mesh=_mesh,
    scratch_types=[
        pltpu.VMEM((BPW,), jnp.int32),
        pltpu.VMEM((CH, D_MODEL), jnp.float32),
        pltpu.SemaphoreType.DMA,
    ],
)
def _emb_lookup(idx_hbm, table_hbm, out_hbm, idx_v, rows_v, gsem):
    wid = lax.axis_index("s") * NC + lax.axis_index("c")
    base = wid * BPW
    pltpu.sync_copy(idx_hbm.at[pl.ds(base, BPW)], idx_v)

    @pl.loop(0, NCHUNK)
    def _chunk(g):
        off = pl.multiple_of(g * CH, CH)
        idx_slice = idx_v.at[pl.ds(off, CH)]
        pltpu.async_copy(table_hbm.at[idx_slice], rows_v, gsem).wait()
        pltpu.sync_copy(rows_v, out_hbm.at[pl.ds(base + off, CH)])


def kernel(x, table):
    idx = x.reshape(NB).astype(jnp.int32)
    out = _emb_lookup(idx, table)
    return out.reshape(BATCH, SEQ_LEN, D_MODEL)


# speedup vs baseline: 2.5394x; 1.1143x over previous
import functools

import jax
import jax.numpy as jnp
from jax import lax
from jax.experimental import pallas as pl
from jax.experimental.pallas import tpu as pltpu
from jax.experimental.pallas import tpu_sc as plsc

SEQ_LEN = 8192
D_MODEL = 768
BATCH = 4

NB = BATCH * SEQ_LEN
NC = 2
NS = 16
NW = NC * NS
BPW = NB // NW
CH = 64
NCHUNK = BPW // CH

_mesh = plsc.VectorSubcoreMesh(core_axis_name="c", subcore_axis_name="s")


@functools.partial(
    pl.kernel,
    out_type=jax.ShapeDtypeStruct((NB, D_MODEL), jnp.float32),
    mesh=_mesh,
    scratch_types=[
        pltpu.VMEM((BPW,), jnp.int32),
        pltpu.VMEM((CH, D_MODEL), jnp.float32),
        pltpu.VMEM((CH, D_MODEL), jnp.float32),
        pltpu.SemaphoreType.DMA,
        pltpu.SemaphoreType.DMA,
    ],
)
def _emb_lookup(idx_hbm, table_hbm, out_hbm, idx_v, buf0, buf1, sem0, sem1):
    wid = lax.axis_index("s") * NC + lax.axis_index("c")
    base = wid * BPW
    pltpu.sync_copy(idx_hbm.at[pl.ds(base, BPW)], idx_v)

    bufs = (buf0, buf1)
    sems = (sem0, sem1)

    def gather(g, j):
        off = pl.multiple_of(g * CH, CH)
        pltpu.async_copy(table_hbm.at[idx_v.at[pl.ds(off, CH)]], bufs[j], sems[j])

    def drain_and_store(g, j):
        pltpu.make_async_copy(table_hbm.at[pl.ds(0, CH)], bufs[j], sems[j]).wait()
        off = pl.multiple_of(g * CH, CH)
        pltpu.sync_copy(bufs[j], out_hbm.at[pl.ds(base + off, CH)])

    gather(0, 0)

    @pl.loop(0, NCHUNK, step=2)
    def _pair(g):
        gather(g + 1, 1)
        drain_and_store(g, 0)

        @pl.when(g + 2 < NCHUNK)
        def _():
            gather(g + 2, 0)

        drain_and_store(g + 1, 1)


def kernel(x, table):
    idx = x.reshape(NB).astype(jnp.int32)
    out = _emb_lookup(idx, table)
    return out.reshape(BATCH, SEQ_LEN, D_MODEL)
